# baseline (speedup 1.0000x reference)
import functools

import jax
import jax.numpy as jnp
from jax import lax
from jax.experimental import pallas as pl
from jax.experimental.pallas import tpu as pltpu
from jax.experimental.pallas import tpu_sc as plsc

ROWS, COLS = 8192, 4096
N = ROWS * COLS
NC, NS = 2, 16
NW = NC * NS
CH = 65536
NCHUNK = N // CH
CPW = NCHUNK // NW
S = 8192
SSH = 13
U = 8
NBND = 520

_mesh = plsc.VectorSubcoreMesh(core_axis_name="c", subcore_axis_name="s")


@functools.partial(
    pl.kernel,
    out_type=jax.ShapeDtypeStruct((N,), jnp.float32),
    mesh=_mesh,
    compiler_params=pltpu.CompilerParams(
        use_tc_tiling_on_sc=False, needs_layout_passes=False),
    scratch_types=[
        pltpu.VMEM((CH,), jnp.float32),
        pltpu.VMEM((2, S), jnp.int32),
        pltpu.VMEM((2, S), jnp.float32),
        pltpu.VMEM((NBND,), jnp.int32),
        pltpu.SemaphoreType.DMA,
        pltpu.SemaphoreType.DMA,
        pltpu.SemaphoreType.DMA,
        pltpu.SemaphoreType.DMA,
        pltpu.SemaphoreType.DMA,
    ],
)
def _apply_sc(x_hbm, sk_hbm, sv_hbm, bnd_hbm, out_hbm,
              acc, kbuf, vbuf, bndb, semx, semk0, semv0, semk1, semv1):
    w = lax.axis_index("s") * NC + lax.axis_index("c")
    pltpu.sync_copy(bnd_hbm, bndb)
    lanes = lax.iota(jnp.int32, 16)
    nextlane = jnp.minimum(lanes + 1, 15)
    is_last_lane = lanes == 15

    def chunk_body(cc, carry):
        c = w * CPW + cc
        base = c * CH
        cx = pltpu.async_copy(x_hbm.at[pl.ds(base, CH)], acc, semx)
        bv = bndb[pl.ds(c, 16)]
        lo = jnp.bitwise_and(bv[0], -8)
        hi = bv[1]
        nb = (hi - lo + (S - 1)) >> SSH

        def off_of(b):
            return pl.multiple_of(jnp.minimum(lo + b * S, N - S), 8)

        def start(b, p, semk, semv):
            off = off_of(b)
            pltpu.async_copy(sk_hbm.at[pl.ds(off, S)], kbuf.at[p], semk)
            pltpu.async_copy(sv_hbm.at[pl.ds(off, S)], vbuf.at[p], semv)

        def wait(b, p, semk, semv):
            off = off_of(b)
            pltpu.make_async_copy(
                sk_hbm.at[pl.ds(off, S)], kbuf.at[p], semk).wait()
            pltpu.make_async_copy(
                sv_hbm.at[pl.ds(off, S)], vbuf.at[p], semv).wait()

        def compute(b, p):
            off = off_of(b)
            vt = (jnp.minimum(hi - off, S) + (16 * U - 1)) >> 7

            def vec_body(v, carry):
                for u in range(U):
                    t = (v * U + u) * 16
                    kv = kbuf[p, pl.ds(t, 16)]
                    vv = vbuf[p, pl.ds(t, 16)]
                    local = kv - base
                    inrange = (kv >= base) & (kv < base + CH)
                    knext = lax.gather(
                        kv, nextlane[:, None],
                        dimension_numbers=lax.GatherDimensionNumbers(
                            offset_dims=(), collapsed_slice_dims=(0,),
                            start_index_map=(0,)),
                        slice_sizes=(1,),
                        mode=lax.GatherScatterMode.PROMISE_IN_BOUNDS)
                    runend = (kv != knext) | is_last_lane
                    plsc.store_scatter(acc, [local], vv,
                                       mask=inrange & runend)
                return carry

            lax.fori_loop(0, vt, vec_body, 0)

        @pl.when(nb > 0)
        def _():
            start(0, 0, semk0, semv0)
        cx.wait()

        def pair_body(t, carry):
            b0 = 2 * t
            wait(b0, 0, semk0, semv0)

            @pl.when(b0 + 1 < nb)
            def _():
                start(b0 + 1, 1, semk1, semv1)
            compute(b0, 0)

            @pl.when(b0 + 1 < nb)
            def _():
                wait(b0 + 1, 1, semk1, semv1)

                @pl.when(b0 + 2 < nb)
                def _():
                    start(b0 + 2, 0, semk0, semv0)
                compute(b0 + 1, 1)
            return carry

        carry = lax.fori_loop(0, (nb + 1) >> 1, pair_body, carry)
        pltpu.sync_copy(acc, out_hbm.at[pl.ds(base, CH)])
        return carry

    lax.fori_loop(0, CPW, chunk_body, 0)


def kernel(x, index, src):
    rows, cols = x.shape
    n = rows * cols
    col = jnp.broadcast_to(jnp.arange(cols, dtype=jnp.int32)[None, :],
                           index.shape)
    lin = index.astype(jnp.int32) * cols + col
    sk, sv = lax.sort((lin.reshape(n), src.reshape(n)), dimension=0,
                      is_stable=False, num_keys=1)
    queries = jnp.minimum(jnp.arange(NBND, dtype=jnp.int32) * CH, n)
    bnd = jnp.searchsorted(sk, queries, side="left").astype(jnp.int32)
    out = _apply_sc(x.reshape(n), sk, sv, bnd)
    return out.reshape(rows, cols)
